# Initial kernel scaffold; baseline (speedup 1.0000x reference)
import jax
import jax.numpy as jnp
from jax.experimental import pallas as pl


def kernel(x_idle, x_quasi, x_task, ei_idle_idle, ei_idle_quasi, ei_quasi_idle, ei_quasi_task, ei_task_quasi, params):
    raise NotImplementedError("write your pallas kernel here")


# speedup vs baseline: 1.0470x; 1.0470x over previous
import functools

import jax
import jax.numpy as jnp
from jax.experimental import pallas as pl
from jax.experimental.pallas import tpu as pltpu

HID = 128
HEADS = 2
OUT_CH = HID // HEADS
N_IDLE = 50000
N_QUASI = 50000
N_TASK = 10000


def _combine_ln_body(a_ref, b_ref, h_ref, g_ref, bt_ref, o_ref):
    x = a_ref[...] + b_ref[...] + h_ref[...]
    mu = jnp.mean(x, axis=-1, keepdims=True)
    xc = x - mu
    var = jnp.mean(xc * xc, axis=-1, keepdims=True)
    y = xc * jax.lax.rsqrt(var + 1e-5) * g_ref[...] + bt_ref[...]
    o_ref[...] = jnp.maximum(y, 0.0)


def _combine_ln(new_a, new_b, h, gamma, beta):
    n = h.shape[0]
    blk = 400
    grid = (n // blk,)
    return pl.pallas_call(
        _combine_ln_body,
        grid=grid,
        in_specs=[
            pl.BlockSpec((blk, HID), lambda i: (i, 0)),
            pl.BlockSpec((blk, HID), lambda i: (i, 0)),
            pl.BlockSpec((blk, HID), lambda i: (i, 0)),
            pl.BlockSpec((1, HID), lambda i: (0, 0)),
            pl.BlockSpec((1, HID), lambda i: (0, 0)),
        ],
        out_specs=pl.BlockSpec((blk, HID), lambda i: (i, 0)),
        out_shape=jax.ShapeDtypeStruct((n, HID), jnp.float32),
    )(new_a, new_b, h, gamma.reshape(1, HID), beta.reshape(1, HID))


def _gat(x_src, x_dst, ei, p, num_dst):
    src, dst = ei[0], ei[1]
    hs = (x_src @ p['W_src']).reshape(-1, HEADS, OUT_CH)
    a_src = (hs * p['att_src'][None]).sum(-1)
    a_dst = x_dst @ (p['W_dst'].reshape(HID, HEADS, OUT_CH) * p['att_dst'][None]).sum(-1)
    alpha = a_src[src] + a_dst[dst]
    alpha = jax.nn.leaky_relu(alpha, 0.2)
    ex = jnp.exp(alpha)
    denom = jax.ops.segment_sum(ex, dst, num_segments=num_dst)
    msg = hs[src] * ex[:, :, None]
    num = jax.ops.segment_sum(msg, dst, num_segments=num_dst)
    out = num / (denom[:, :, None] + 1e-16)
    return out.reshape(num_dst, HID) + p['bias']


def kernel(x_idle, x_quasi, x_task, ei_idle_idle, ei_idle_quasi, ei_quasi_idle, ei_quasi_task, ei_task_quasi, params):
    h = {}
    h['idle'] = jax.nn.relu(x_idle @ params['lin']['idle']['W'] + params['lin']['idle']['b'])
    h['quasi'] = jax.nn.relu(x_quasi @ params['lin']['quasi']['W'] + params['lin']['quasi']['b'])
    h['task'] = jax.nn.relu(x_task @ params['lin']['task']['W'] + params['lin']['task']['b'])
    for layer in params['layers']:
        c = layer['conv']
        gi_a = _gat(h['idle'], h['idle'], ei_idle_idle, c['idle__idle'], N_IDLE)
        gi_b = _gat(h['quasi'], h['idle'], ei_quasi_idle, c['quasi__idle'], N_IDLE)
        gq_a = _gat(h['idle'], h['quasi'], ei_idle_quasi, c['idle__quasi'], N_QUASI)
        gq_b = _gat(h['task'], h['quasi'], ei_task_quasi, c['task__quasi'], N_QUASI)
        gt_a = _gat(h['quasi'], h['task'], ei_quasi_task, c['quasi__task'], N_TASK)
        nrm = layer['norm']
        h['idle'] = _combine_ln(gi_a, gi_b, h['idle'], nrm['idle']['gamma'], nrm['idle']['beta'])
        h['quasi'] = _combine_ln(gq_a, gq_b, h['quasi'], nrm['quasi']['gamma'], nrm['quasi']['beta'])
        h['task'] = _combine_ln(gt_a, jnp.zeros_like(gt_a), h['task'], nrm['task']['gamma'], nrm['task']['beta'])
    return (h['idle'], h['quasi'], h['task'])


# speedup vs baseline: 18.0068x; 17.1989x over previous
import functools

import jax
import jax.numpy as jnp
from jax import lax
from jax.experimental import pallas as pl
from jax.experimental.pallas import tpu as pltpu
from jax.experimental.pallas import tpu_sc as plsc

HID = 128
HEADS = 2
OUT_CH = HID // HEADS
N_IDLE = 50000
N_QUASI = 50000
N_TASK = 10000
E = 100000

NC, NS, L = 2, 16, 16
NW = NC * NS
EW = 3136
EP = NW * EW
NVE = EW // L
BLK = 224
NBLK = EW // BLK
CMAX = 7168


def _rup(n, m):
    return (n + m - 1) // m * m


def _edge_body(n_src, n_dst, ndp, nchunk, csize,
               src_h, dst_h, as0_h, as1_h, ad0_h, ad1_h, hs_h,
               num_h, den0_h, den1_h,
               src_v, dst_v, g0, g1, g2, g3, ex0_v, ex1_v,
               exb0, exb1, gblk, sblk, msg_v, zb_v, zb2_v, sem,
               num_sp, den0_sp, den1_sp):
    c = lax.axis_index("c")
    s = lax.axis_index("s")
    w = c * NS + s
    base_e = w * EW

    pltpu.sync_copy(src_h.at[pl.ds(base_e, EW)], src_v)
    pltpu.sync_copy(dst_h.at[pl.ds(base_e, EW)], dst_v)

    pltpu.async_copy(as0_h.at[src_v], g0, sem).wait()
    pltpu.async_copy(as1_h.at[src_v], g1, sem).wait()
    pltpu.async_copy(ad0_h.at[dst_v], g2, sem).wait()
    pltpu.async_copy(ad1_h.at[dst_v], g3, sem).wait()

    def _z16(i, _):
        zb_v[pl.ds(i * L, L)] = jnp.zeros((L,), jnp.float32)
        return 0
    lax.fori_loop(0, EW // L, _z16, 0)

    def _z2(i, _):
        zb2_v[pl.ds(i * 8, 8), :] = jnp.zeros((8, HID), jnp.float32)
        return 0
    lax.fori_loop(0, 56 // 8, _z2, 0)

    dslice = ndp // NS
    pltpu.sync_copy(zb_v.at[pl.ds(0, dslice)], den0_sp.at[pl.ds(s * dslice, dslice)])
    pltpu.sync_copy(zb_v.at[pl.ds(0, dslice)], den1_sp.at[pl.ds(s * dslice, dslice)])

    def _ex(i, _):
        sl = pl.ds(i * L, L)
        valid = (jnp.full((L,), i * L, jnp.int32) + lax.iota(jnp.int32, L)
                 + base_e) < E
        a0 = g0[sl] + g2[sl]
        a1 = g1[sl] + g3[sl]
        a0 = jnp.where(a0 >= 0.0, a0, a0 * 0.2)
        a1 = jnp.where(a1 >= 0.0, a1, a1 * 0.2)
        ex0_v[sl] = jnp.where(valid, jnp.exp(a0), 0.0)
        ex1_v[sl] = jnp.where(valid, jnp.exp(a1), 0.0)
        return 0
    lax.fori_loop(0, NVE, _ex, 0)

    plsc.subcore_barrier()
    pltpu.sync_copy(ex0_v, den0_sp.at[dst_v], add=True)
    pltpu.sync_copy(ex1_v, den1_sp.at[dst_v], add=True)
    plsc.subcore_barrier()
    pltpu.sync_copy(den0_sp.at[pl.ds(s * dslice, dslice)], g0.at[pl.ds(0, dslice)])
    pltpu.sync_copy(g0.at[pl.ds(0, dslice)],
                    den0_h.at[pl.ds(c * ndp + s * dslice, dslice)])
    pltpu.sync_copy(den1_sp.at[pl.ds(s * dslice, dslice)], g1.at[pl.ds(0, dslice)])
    pltpu.sync_copy(g1.at[pl.ds(0, dslice)],
                    den1_h.at[pl.ds(c * ndp + s * dslice, dslice)])

    for ci in range(nchunk):
        cbase = ci * csize
        rows_out = min(csize, ndp - cbase)
        rpt = csize // NS
        plsc.subcore_barrier()

        def _zc(j, _):
            pltpu.sync_copy(zb2_v, num_sp.at[pl.ds(s * rpt + j * 56, 56), :])
            return 0
        lax.fori_loop(0, rpt // 56, _zc, 0)
        plsc.subcore_barrier()

        def _blk(k, _):
            off = k * BLK

            def _prep(i, _):
                sl16 = pl.ds(i * L, L)
                sle = pl.ds(off + i * L, L)
                gblk[sl16] = src_v[sle]
                dv = dst_v[sle] - cbase
                inc = (dv >= 0) & (dv < csize)
                sblk[sl16] = jnp.where(inc, dv, 0)
                exb0[sl16] = jnp.where(inc, ex0_v[sle], 0.0)
                exb1[sl16] = jnp.where(inc, ex1_v[sle], 0.0)
                return 0
            lax.fori_loop(0, BLK // L, _prep, 0)

            pltpu.async_copy(hs_h.at[gblk], msg_v, sem).wait()

            def _mul(i, _):
                ev0 = exb0[pl.ds(i * L, L)]
                ev1 = exb1[pl.ds(i * L, L)]
                for lane in range(L):
                    e = i * L + lane
                    s0 = ev0[lane]
                    s1 = ev1[lane]
                    for cg in range(4):
                        sl = pl.ds(cg * L, L)
                        msg_v[e, sl] = msg_v[e, sl] * s0
                    for cg in range(4, 8):
                        sl = pl.ds(cg * L, L)
                        msg_v[e, sl] = msg_v[e, sl] * s1
                return 0
            lax.fori_loop(0, BLK // L, _mul, 0)

            pltpu.sync_copy(msg_v, num_sp.at[sblk], add=True)
            return 0
        lax.fori_loop(0, NBLK, _blk, 0)

        plsc.subcore_barrier()
        rpo = rows_out // NS
        pltpu.sync_copy(
            num_sp.at[pl.ds(s * rpo, rpo), :],
            num_h.at[pl.ds(c * ndp + cbase + s * rpo, rpo), :])


@functools.partial(jax.jit, static_argnums=(7, 8))
def _edge_phase(src_p, dst_p, as0, as1, ad0, ad1, hs, n_src, n_dst):
    ndp = _rup(n_dst, NS * 56)
    csize = min(CMAX, ndp)
    nchunk = -(-ndp // csize)
    body = functools.partial(_edge_body, n_src, n_dst, ndp, nchunk, csize)
    mesh = plsc.VectorSubcoreMesh(core_axis_name="c", subcore_axis_name="s")
    f = pl.kernel(
        body,
        out_type=[
            jax.ShapeDtypeStruct((NC * ndp, HID), jnp.float32),
            jax.ShapeDtypeStruct((NC * ndp,), jnp.float32),
            jax.ShapeDtypeStruct((NC * ndp,), jnp.float32),
        ],
        mesh=mesh,
        scratch_types=[
            pltpu.VMEM((EW,), jnp.int32),
            pltpu.VMEM((EW,), jnp.int32),
            pltpu.VMEM((EW,), jnp.float32),
            pltpu.VMEM((EW,), jnp.float32),
            pltpu.VMEM((EW,), jnp.float32),
            pltpu.VMEM((EW,), jnp.float32),
            pltpu.VMEM((EW,), jnp.float32),
            pltpu.VMEM((EW,), jnp.float32),
            pltpu.VMEM((BLK,), jnp.float32),
            pltpu.VMEM((BLK,), jnp.float32),
            pltpu.VMEM((BLK,), jnp.int32),
            pltpu.VMEM((BLK,), jnp.int32),
            pltpu.VMEM((BLK, HID), jnp.float32),
            pltpu.VMEM((EW,), jnp.float32),
            pltpu.VMEM((56, HID), jnp.float32),
            pltpu.SemaphoreType.DMA,
            pltpu.VMEM_SHARED((csize, HID), jnp.float32),
            pltpu.VMEM_SHARED((ndp,), jnp.float32),
            pltpu.VMEM_SHARED((ndp,), jnp.float32),
        ],
    )
    num, den0, den1 = f(src_p, dst_p, as0, as1, ad0, ad1, hs)
    num = num[:ndp][:n_dst] + num[ndp:][:n_dst]
    den0 = den0[:ndp][:n_dst] + den0[ndp:][:n_dst]
    den1 = den1[:ndp][:n_dst] + den1[ndp:][:n_dst]
    return num, den0, den1


def _combine_ln_body(a_ref, b_ref, h_ref, g_ref, bt_ref, o_ref):
    x = a_ref[...] + b_ref[...] + h_ref[...]
    mu = jnp.mean(x, axis=-1, keepdims=True)
    xc = x - mu
    var = jnp.mean(xc * xc, axis=-1, keepdims=True)
    y = xc * jax.lax.rsqrt(var + 1e-5) * g_ref[...] + bt_ref[...]
    o_ref[...] = jnp.maximum(y, 0.0)


def _combine_ln(new_a, new_b, h, gamma, beta):
    n = h.shape[0]
    blk = 400
    grid = (n // blk,)
    return pl.pallas_call(
        _combine_ln_body,
        grid=grid,
        in_specs=[
            pl.BlockSpec((blk, HID), lambda i: (i, 0)),
            pl.BlockSpec((blk, HID), lambda i: (i, 0)),
            pl.BlockSpec((blk, HID), lambda i: (i, 0)),
            pl.BlockSpec((1, HID), lambda i: (0, 0)),
            pl.BlockSpec((1, HID), lambda i: (0, 0)),
        ],
        out_specs=pl.BlockSpec((blk, HID), lambda i: (i, 0)),
        out_shape=jax.ShapeDtypeStruct((n, HID), jnp.float32),
    )(new_a, new_b, h, gamma.reshape(1, HID), beta.reshape(1, HID))


def _gat(x_src, x_dst, eis, p, num_dst):
    src_p, dst_p = eis
    n_src = x_src.shape[0]
    hs = x_src @ p['W_src']
    att_mat = jnp.zeros((HID, HEADS), jnp.float32)
    att_mat = att_mat.at[:OUT_CH, 0].set(p['att_src'][0])
    att_mat = att_mat.at[OUT_CH:, 1].set(p['att_src'][1])
    a_src = hs @ att_mat
    vmat = (p['W_dst'].reshape(HID, HEADS, OUT_CH) * p['att_dst'][None]).sum(-1)
    a_dst = x_dst @ vmat
    num, den0, den1 = _edge_phase(
        src_p, dst_p,
        a_src[:, 0].copy(), a_src[:, 1].copy(),
        a_dst[:, 0].copy(), a_dst[:, 1].copy(),
        hs, n_src, num_dst)
    den = jnp.stack([den0, den1], axis=1)
    w = 1.0 / (den + 1e-16)
    wfull = jnp.repeat(w, OUT_CH, axis=1)
    return num * wfull + p['bias']


def _pad_ei(ei):
    return (jnp.pad(ei[0], (0, EP - E)), jnp.pad(ei[1], (0, EP - E)))


def kernel(x_idle, x_quasi, x_task, ei_idle_idle, ei_idle_quasi, ei_quasi_idle, ei_quasi_task, ei_task_quasi, params):
    e_ii = _pad_ei(ei_idle_idle)
    e_iq = _pad_ei(ei_idle_quasi)
    e_qi = _pad_ei(ei_quasi_idle)
    e_qt = _pad_ei(ei_quasi_task)
    e_tq = _pad_ei(ei_task_quasi)
    h = {}
    h['idle'] = jax.nn.relu(x_idle @ params['lin']['idle']['W'] + params['lin']['idle']['b'])
    h['quasi'] = jax.nn.relu(x_quasi @ params['lin']['quasi']['W'] + params['lin']['quasi']['b'])
    h['task'] = jax.nn.relu(x_task @ params['lin']['task']['W'] + params['lin']['task']['b'])
    for layer in params['layers']:
        c = layer['conv']
        gi_a = _gat(h['idle'], h['idle'], e_ii, c['idle__idle'], N_IDLE)
        gi_b = _gat(h['quasi'], h['idle'], e_qi, c['quasi__idle'], N_IDLE)
        gq_a = _gat(h['idle'], h['quasi'], e_iq, c['idle__quasi'], N_QUASI)
        gq_b = _gat(h['task'], h['quasi'], e_tq, c['task__quasi'], N_QUASI)
        gt_a = _gat(h['quasi'], h['task'], e_qt, c['quasi__task'], N_TASK)
        nrm = layer['norm']
        h['idle'] = _combine_ln(gi_a, gi_b, h['idle'], nrm['idle']['gamma'], nrm['idle']['beta'])
        h['quasi'] = _combine_ln(gq_a, gq_b, h['quasi'], nrm['quasi']['gamma'], nrm['quasi']['beta'])
        h['task'] = _combine_ln(gt_a, jnp.zeros_like(gt_a), h['task'], nrm['task']['gamma'], nrm['task']['beta'])
    return (h['idle'], h['quasi'], h['task'])
